# baseline (speedup 1.0000x reference)
import functools

import jax
import jax.numpy as jnp
from jax import lax
from jax.experimental import pallas as pl
from jax.experimental.pallas import tpu as pltpu
from jax.experimental.pallas import tpu_sc as plsc

_T = 2048
_D = 2048
_E = 8
_K = 2
_I = 1024
_NORM_MIN = 1e-12

_N = _T * _K
_BR = 256
_NP = _N + _E * _BR
_NT = _NP // _BR
_BT = 256
_LANES = 128


def _router_body(x_ref, wr_ref, b_ref, logits_ref, i1_ref, i2_ref,
                 w1_ref, w2_ref):
    x = x_ref[...]
    wr = wr_ref[...]
    logits = jnp.dot(x, wr, preferred_element_type=jnp.float32)
    logits_ref[...] = logits
    lane = lax.broadcasted_iota(jnp.int32, logits.shape, 1)
    valid = lane < _E
    neg = jnp.float32(-1e30)
    ml = jnp.where(valid, logits, neg)
    m = jnp.max(ml, axis=1, keepdims=True)
    p = jnp.where(valid, jnp.exp(ml - m), 0.0)
    probs = p / jnp.sum(p, axis=1, keepdims=True)
    biased = jnp.where(valid, probs + b_ref[0:1, :], neg)
    big = jnp.int32(1 << 30)
    m1 = jnp.max(biased, axis=1, keepdims=True)
    i1 = jnp.min(jnp.where(biased == m1, lane, big), axis=1, keepdims=True)
    sel1 = lane == i1
    biased2 = jnp.where(sel1, neg, biased)
    m2 = jnp.max(biased2, axis=1, keepdims=True)
    i2 = jnp.min(jnp.where(biased2 == m2, lane, big), axis=1, keepdims=True)
    sel2 = lane == i2
    w1 = jnp.sum(jnp.where(sel1, probs, 0.0), axis=1, keepdims=True)
    w2 = jnp.sum(jnp.where(sel2, probs, 0.0), axis=1, keepdims=True)
    denom = jnp.maximum(w1 + w2, _NORM_MIN)
    shp = logits.shape
    i1_ref[...] = jnp.broadcast_to(i1, shp)
    i2_ref[...] = jnp.broadcast_to(i2, shp)
    w1_ref[...] = jnp.broadcast_to(w1 / denom, shp)
    w2_ref[...] = jnp.broadcast_to(w2 / denom, shp)


def _run_router(hs, router_weight, bias):
    wr_pad = jnp.zeros((_D, _LANES), jnp.float32).at[:, :_E].set(
        router_weight.astype(jnp.float32).T)
    b_pad = jnp.zeros((8, _LANES), jnp.float32).at[:, :_E].set(
        jnp.broadcast_to(bias.reshape(1, _E), (8, _E)))
    f32 = jnp.float32
    outs = pl.pallas_call(
        _router_body,
        grid=(_T // _BT,),
        in_specs=[
            pl.BlockSpec((_BT, _D), lambda i: (i, 0)),
            pl.BlockSpec((_D, _LANES), lambda i: (0, 0)),
            pl.BlockSpec((8, _LANES), lambda i: (0, 0)),
        ],
        out_specs=[
            pl.BlockSpec((_BT, _LANES), lambda i: (i, 0)),
            pl.BlockSpec((_BT, _LANES), lambda i: (i, 0)),
            pl.BlockSpec((_BT, _LANES), lambda i: (i, 0)),
            pl.BlockSpec((_BT, _LANES), lambda i: (i, 0)),
            pl.BlockSpec((_BT, _LANES), lambda i: (i, 0)),
        ],
        out_shape=[
            jax.ShapeDtypeStruct((_T, _LANES), f32),
            jax.ShapeDtypeStruct((_T, _LANES), jnp.int32),
            jax.ShapeDtypeStruct((_T, _LANES), jnp.int32),
            jax.ShapeDtypeStruct((_T, _LANES), f32),
            jax.ShapeDtypeStruct((_T, _LANES), f32),
        ],
    )(hs, wr_pad, b_pad)
    logits_p, i1, i2, w1, w2 = outs
    return logits_p[:, :_E], i1[:, 0], i2[:, 0], w1[:, 0], w2[:, 0]


def _route_metadata(i1, i2, w1, w2):
    i32 = jnp.int32
    expert_a = jnp.stack([i1, i2], axis=1).reshape(-1)
    w_a = jnp.stack([w1, w2], axis=1).reshape(-1)
    tok_a = jnp.arange(_N, dtype=i32) // _K
    order = jnp.argsort(expert_a, stable=True)
    sorted_e = expert_a[order]
    sorted_tok = tok_a[order]
    sorted_w = w_a[order]
    gs = jnp.zeros((_E,), i32).at[expert_a].add(1)
    tiles_e = (gs + _BR - 1) // _BR
    tile_start = jnp.concatenate([jnp.zeros((1,), i32),
                                  jnp.cumsum(tiles_e)[:-1]])
    total_tiles = jnp.sum(tiles_e)
    grp_start = jnp.concatenate([jnp.zeros((1,), i32),
                                 jnp.cumsum(gs)[:-1]])
    row_start = tile_start * _BR
    p = row_start[sorted_e] + jnp.arange(_N, dtype=i32) - grp_start[sorted_e]
    padded_tok = jnp.zeros((_NP,), i32).at[p].set(sorted_tok)
    w_pad = jnp.zeros((_NP,), jnp.float32).at[p].set(sorted_w)
    inv = jnp.zeros((_N,), i32).at[order].set(jnp.arange(_N, dtype=i32))
    p_of_a = p[inv]
    pos0 = p_of_a[0::2]
    pos1 = p_of_a[1::2]
    t_ids = jnp.arange(_NT, dtype=i32)
    te_raw = jnp.searchsorted(tile_start, t_ids, side='right').astype(i32) - 1
    last = total_tiles - 1
    valid = (t_ids < total_tiles).astype(i32)
    tile_expert = jnp.where(valid == 1, te_raw, te_raw[last])
    rowblk = jnp.where(valid == 1, t_ids, last)
    meta = jnp.stack([valid, tile_expert, rowblk])
    return padded_tok, w_pad, pos0, pos1, meta


def _gmm_body(m_ref, x_ref, g_ref, u_ref, d_ref, wrow_ref, y_ref):
    t = pl.program_id(0)

    @pl.when(m_ref[0, t] == 1)
    def _():
        x = x_ref[...].astype(jnp.bfloat16)
        gate = jnp.dot(x, g_ref[0].astype(jnp.bfloat16),
                       preferred_element_type=jnp.float32)
        up = jnp.dot(x, u_ref[0].astype(jnp.bfloat16),
                     preferred_element_type=jnp.float32)
        act = (gate * lax.logistic(gate)) * up
        contrib = jnp.dot(act.astype(jnp.bfloat16),
                          d_ref[0].astype(jnp.bfloat16),
                          preferred_element_type=jnp.float32)
        y_ref[...] = contrib * wrow_ref[...][:, 0:1]


def _run_gmm(x_pad, gate_up_proj, down_proj, w_pad, meta):
    w_pad2 = jnp.broadcast_to(w_pad[:, None], (_NP, _LANES))
    grid_spec = pltpu.PrefetchScalarGridSpec(
        num_scalar_prefetch=1,
        grid=(_NT,),
        in_specs=[
            pl.BlockSpec((_BR, _D), lambda t, m: (m[2, t], 0)),
            pl.BlockSpec((1, _D, _I), lambda t, m: (m[1, t], 0, 0)),
            pl.BlockSpec((1, _D, _I), lambda t, m: (m[1, t], 0, 1)),
            pl.BlockSpec((1, _I, _D), lambda t, m: (m[1, t], 0, 0)),
            pl.BlockSpec((_BR, _LANES), lambda t, m: (m[2, t], 0)),
        ],
        out_specs=pl.BlockSpec((_BR, _D), lambda t, m: (m[2, t], 0)),
    )
    return pl.pallas_call(
        _gmm_body,
        grid_spec=grid_spec,
        out_shape=jax.ShapeDtypeStruct((_NP, _D), jnp.float32),
        compiler_params=pltpu.CompilerParams(
            dimension_semantics=("arbitrary",)),
    )(meta, x_pad, gate_up_proj, gate_up_proj, down_proj, w_pad2)


_NW = 32
_GCH = 16


def _sc_mesh():
    return plsc.VectorSubcoreMesh(core_axis_name="c", subcore_axis_name="s")


def _dispatch(hs, padded_tok):
    rows_per = _NP // _NW
    ch = 24
    nch = rows_per // ch

    @functools.partial(
        pl.kernel, mesh=_sc_mesh(),
        out_type=jax.ShapeDtypeStruct((_NP, _D), jnp.float32),
        scratch_types=[
            pltpu.VMEM((rows_per,), jnp.int32),
            pltpu.VMEM((ch, _D), jnp.float32),
            pltpu.VMEM((ch, _D), jnp.float32),
            pltpu.SemaphoreType.DMA,
            pltpu.SemaphoreType.DMA,
        ],
    )
    def k(hs_hbm, tok_hbm, out_hbm, idx_v, buf0, buf1, s0, s1):
        wid = lax.axis_index("s") * 2 + lax.axis_index("c")
        base = wid * rows_per
        pltpu.sync_copy(tok_hbm.at[pl.ds(base, rows_per)], idx_v)
        bufs = (buf0, buf1)
        sems = (s0, s1)
        pltpu.async_copy(hs_hbm.at[idx_v.at[pl.ds(0, ch)]], buf0, s0)
        for c in range(nch):
            if c + 1 < nch:
                pltpu.async_copy(
                    hs_hbm.at[idx_v.at[pl.ds((c + 1) * ch, ch)]],
                    bufs[(c + 1) % 2], sems[(c + 1) % 2])
            pltpu.make_async_copy(
                hs_hbm.at[idx_v.at[pl.ds(c * ch, ch)]],
                bufs[c % 2], sems[c % 2]).wait()
            pltpu.sync_copy(bufs[c % 2], out_hbm.at[pl.ds(base + c * ch, ch)])

    return k(hs, padded_tok)


def _combine(y_pad, pos0, pos1):
    rows_per = _T // _NW
    ch = 8
    nch = rows_per // ch

    @functools.partial(
        pl.kernel, mesh=_sc_mesh(),
        out_type=jax.ShapeDtypeStruct((_T, _D), jnp.float32),
        scratch_types=[
            pltpu.VMEM((rows_per,), jnp.int32),
            pltpu.VMEM((rows_per,), jnp.int32),
            pltpu.VMEM((ch, _D), jnp.float32),
            pltpu.VMEM((ch, _D), jnp.float32),
            pltpu.VMEM((ch, _D), jnp.float32),
            pltpu.VMEM((ch, _D), jnp.float32),
            pltpu.SemaphoreType.DMA,
            pltpu.SemaphoreType.DMA,
            pltpu.SemaphoreType.DMA,
            pltpu.SemaphoreType.DMA,
        ],
    )
    def k(y_hbm, p0_hbm, p1_hbm, out_hbm, i0_v, i1_v, b0a, b1a, b0b, b1b,
          s0a, s1a, s0b, s1b):
        wid = lax.axis_index("s") * 2 + lax.axis_index("c")
        base = wid * rows_per
        pltpu.sync_copy(p0_hbm.at[pl.ds(base, rows_per)], i0_v)
        pltpu.sync_copy(p1_hbm.at[pl.ds(base, rows_per)], i1_v)
        b0 = (b0a, b0b)
        b1 = (b1a, b1b)
        s0 = (s0a, s0b)
        s1 = (s1a, s1b)

        def fire(c):
            pltpu.async_copy(y_hbm.at[i0_v.at[pl.ds(c * ch, ch)]],
                             b0[c % 2], s0[c % 2])
            pltpu.async_copy(y_hbm.at[i1_v.at[pl.ds(c * ch, ch)]],
                             b1[c % 2], s1[c % 2])

        fire(0)
        for c in range(nch):
            if c + 1 < nch:
                fire(c + 1)
            pltpu.make_async_copy(y_hbm.at[i0_v.at[pl.ds(c * ch, ch)]],
                                  b0[c % 2], s0[c % 2]).wait()
            pltpu.make_async_copy(y_hbm.at[i1_v.at[pl.ds(c * ch, ch)]],
                                  b1[c % 2], s1[c % 2]).wait()

            for r in range(ch):
                def addsl(j, c2, r=r):
                    sl = pl.ds(j * 16, 16)
                    b0[c % 2][r, sl] = b0[c % 2][r, sl] + b1[c % 2][r, sl]
                    return c2

                lax.fori_loop(0, _D // 16, addsl, 0)
            pltpu.sync_copy(b0[c % 2], out_hbm.at[pl.ds(base + c * ch, ch)])

    return k(y_pad, pos0, pos1)


def kernel(hidden_states, router_weight, e_score_correction_bias,
           gate_up_proj, down_proj):
    assert hidden_states.shape == (_T, _D)
    assert gate_up_proj.shape == (_E, _D, 2 * _I)
    hs = hidden_states.astype(jnp.float32)
    logits, i1, i2, w1, w2 = _run_router(hs, router_weight,
                                         e_score_correction_bias)
    padded_tok, w_pad, pos0, pos1, meta = _route_metadata(i1, i2, w1, w2)
    x_pad = _dispatch(hs, padded_tok)
    y_pad = _run_gmm(x_pad, gate_up_proj.astype(jnp.float32),
                     down_proj.astype(jnp.float32), w_pad, meta)
    final = _combine(y_pad, pos0, pos1)
    return final.reshape(-1), logits.reshape(-1)


# speedup vs baseline: 1.4899x; 1.4899x over previous
import functools

import jax
import jax.numpy as jnp
from jax import lax
from jax.experimental import pallas as pl
from jax.experimental.pallas import tpu as pltpu
from jax.experimental.pallas import tpu_sc as plsc

_T = 2048
_D = 2048
_E = 8
_K = 2
_I = 1024
_NORM_MIN = 1e-12

_N = _T * _K
_BR = 256
_NP = _N + _E * _BR
_NT = _NP // _BR
_BT = 256
_LANES = 128


def _router_body(x_ref, wr_ref, b_ref, logits_ref, i1_ref, i2_ref,
                 w1_ref, w2_ref):
    x = x_ref[...]
    wr = wr_ref[...]
    logits = jnp.dot(x, wr, preferred_element_type=jnp.float32)
    logits_ref[...] = logits
    lane = lax.broadcasted_iota(jnp.int32, logits.shape, 1)
    valid = lane < _E
    neg = jnp.float32(-1e30)
    ml = jnp.where(valid, logits, neg)
    m = jnp.max(ml, axis=1, keepdims=True)
    p = jnp.where(valid, jnp.exp(ml - m), 0.0)
    probs = p / jnp.sum(p, axis=1, keepdims=True)
    biased = jnp.where(valid, probs + b_ref[0:1, :], neg)
    big = jnp.int32(1 << 30)
    m1 = jnp.max(biased, axis=1, keepdims=True)
    i1 = jnp.min(jnp.where(biased == m1, lane, big), axis=1, keepdims=True)
    sel1 = lane == i1
    biased2 = jnp.where(sel1, neg, biased)
    m2 = jnp.max(biased2, axis=1, keepdims=True)
    i2 = jnp.min(jnp.where(biased2 == m2, lane, big), axis=1, keepdims=True)
    sel2 = lane == i2
    w1 = jnp.sum(jnp.where(sel1, probs, 0.0), axis=1, keepdims=True)
    w2 = jnp.sum(jnp.where(sel2, probs, 0.0), axis=1, keepdims=True)
    denom = jnp.maximum(w1 + w2, _NORM_MIN)
    shp = logits.shape
    i1_ref[...] = jnp.broadcast_to(i1, shp)
    i2_ref[...] = jnp.broadcast_to(i2, shp)
    w1_ref[...] = jnp.broadcast_to(w1 / denom, shp)
    w2_ref[...] = jnp.broadcast_to(w2 / denom, shp)


def _run_router(hs, router_weight, bias):
    wr_pad = jnp.zeros((_D, _LANES), jnp.float32).at[:, :_E].set(
        router_weight.astype(jnp.float32).T)
    b_pad = jnp.zeros((8, _LANES), jnp.float32).at[:, :_E].set(
        jnp.broadcast_to(bias.reshape(1, _E), (8, _E)))
    f32 = jnp.float32
    outs = pl.pallas_call(
        _router_body,
        grid=(_T // _BT,),
        in_specs=[
            pl.BlockSpec((_BT, _D), lambda i: (i, 0)),
            pl.BlockSpec((_D, _LANES), lambda i: (0, 0)),
            pl.BlockSpec((8, _LANES), lambda i: (0, 0)),
        ],
        out_specs=[
            pl.BlockSpec((_BT, _LANES), lambda i: (i, 0)),
            pl.BlockSpec((_BT, _LANES), lambda i: (i, 0)),
            pl.BlockSpec((_BT, _LANES), lambda i: (i, 0)),
            pl.BlockSpec((_BT, _LANES), lambda i: (i, 0)),
            pl.BlockSpec((_BT, _LANES), lambda i: (i, 0)),
        ],
        out_shape=[
            jax.ShapeDtypeStruct((_T, _LANES), f32),
            jax.ShapeDtypeStruct((_T, _LANES), jnp.int32),
            jax.ShapeDtypeStruct((_T, _LANES), jnp.int32),
            jax.ShapeDtypeStruct((_T, _LANES), f32),
            jax.ShapeDtypeStruct((_T, _LANES), f32),
        ],
    )(hs, wr_pad, b_pad)
    logits_p, i1, i2, w1, w2 = outs
    return logits_p[:, :_E], i1[:, 0], i2[:, 0], w1[:, 0], w2[:, 0]


def _route_metadata(i1, i2, w1, w2):
    i32 = jnp.int32
    expert_a = jnp.stack([i1, i2], axis=1).reshape(-1)
    w_a = jnp.stack([w1, w2], axis=1).reshape(-1)
    tok_a = jnp.arange(_N, dtype=i32) // _K
    order = jnp.argsort(expert_a, stable=True)
    sorted_e = expert_a[order]
    sorted_tok = tok_a[order]
    sorted_w = w_a[order]
    gs = jnp.zeros((_E,), i32).at[expert_a].add(1)
    tiles_e = (gs + _BR - 1) // _BR
    tile_start = jnp.concatenate([jnp.zeros((1,), i32),
                                  jnp.cumsum(tiles_e)[:-1]])
    total_tiles = jnp.sum(tiles_e)
    grp_start = jnp.concatenate([jnp.zeros((1,), i32),
                                 jnp.cumsum(gs)[:-1]])
    row_start = tile_start * _BR
    p = row_start[sorted_e] + jnp.arange(_N, dtype=i32) - grp_start[sorted_e]
    padded_tok = (jnp.arange(_NP, dtype=i32) % _T).at[p].set(sorted_tok)
    w_pad = jnp.zeros((_NP,), jnp.float32).at[p].set(sorted_w)
    inv = jnp.zeros((_N,), i32).at[order].set(jnp.arange(_N, dtype=i32))
    p_of_a = p[inv]
    pos0 = p_of_a[0::2]
    pos1 = p_of_a[1::2]
    t_ids = jnp.arange(_NT, dtype=i32)
    te_raw = jnp.searchsorted(tile_start, t_ids, side='right').astype(i32) - 1
    last = total_tiles - 1
    valid = (t_ids < total_tiles).astype(i32)
    tile_expert = jnp.where(valid == 1, te_raw, te_raw[last])
    rowblk = jnp.where(valid == 1, t_ids, last)
    meta = jnp.stack([valid, tile_expert, rowblk])
    return padded_tok, w_pad, pos0, pos1, meta


def _gmm_body(m_ref, x_ref, g_ref, u_ref, d_ref, wrow_ref, y_ref):
    t = pl.program_id(0)

    @pl.when(m_ref[0, t] == 1)
    def _():
        x = x_ref[...].astype(jnp.bfloat16)
        gate = jnp.dot(x, g_ref[0].astype(jnp.bfloat16),
                       preferred_element_type=jnp.float32)
        up = jnp.dot(x, u_ref[0].astype(jnp.bfloat16),
                     preferred_element_type=jnp.float32)
        act = (gate * lax.logistic(gate)) * up
        contrib = jnp.dot(act.astype(jnp.bfloat16),
                          d_ref[0].astype(jnp.bfloat16),
                          preferred_element_type=jnp.float32)
        y_ref[...] = contrib * wrow_ref[...][:, 0:1]


def _run_gmm(x_pad, gate_up_proj, down_proj, w_pad, meta):
    w_pad2 = jnp.broadcast_to(w_pad[:, None], (_NP, _LANES))
    grid_spec = pltpu.PrefetchScalarGridSpec(
        num_scalar_prefetch=1,
        grid=(_NT,),
        in_specs=[
            pl.BlockSpec((_BR, _D), lambda t, m: (m[2, t], 0)),
            pl.BlockSpec((1, _D, _I), lambda t, m: (m[1, t], 0, 0)),
            pl.BlockSpec((1, _D, _I), lambda t, m: (m[1, t], 0, 1)),
            pl.BlockSpec((1, _I, _D), lambda t, m: (m[1, t], 0, 0)),
            pl.BlockSpec((_BR, _LANES), lambda t, m: (m[2, t], 0)),
        ],
        out_specs=pl.BlockSpec((_BR, _D), lambda t, m: (m[2, t], 0)),
    )
    return pl.pallas_call(
        _gmm_body,
        grid_spec=grid_spec,
        out_shape=jax.ShapeDtypeStruct((_NP, _D), jnp.float32),
        compiler_params=pltpu.CompilerParams(
            dimension_semantics=("arbitrary",)),
    )(meta, x_pad, gate_up_proj, gate_up_proj, down_proj, w_pad2)


_NW = 32
_GCH = 16


def _sc_mesh():
    return plsc.VectorSubcoreMesh(core_axis_name="c", subcore_axis_name="s")


def _dispatch(hs, padded_tok):
    rows_per = _NP // _NW
    ch = 24
    nch = rows_per // ch

    @functools.partial(
        pl.kernel, mesh=_sc_mesh(),
        out_type=jax.ShapeDtypeStruct((_NP, _D), jnp.float32),
        scratch_types=[
            pltpu.VMEM((rows_per,), jnp.int32),
            pltpu.VMEM((ch, _D), jnp.float32),
            pltpu.VMEM((ch, _D), jnp.float32),
            pltpu.SemaphoreType.DMA,
            pltpu.SemaphoreType.DMA,
        ],
    )
    def k(hs_hbm, tok_hbm, out_hbm, idx_v, buf0, buf1, s0, s1):
        wid = lax.axis_index("s") * 2 + lax.axis_index("c")
        base = wid * rows_per
        pltpu.sync_copy(tok_hbm.at[pl.ds(base, rows_per)], idx_v)
        bufs = (buf0, buf1)
        sems = (s0, s1)
        pltpu.async_copy(hs_hbm.at[idx_v.at[pl.ds(0, ch)]], buf0, s0)
        for c in range(nch):
            if c + 1 < nch:
                pltpu.async_copy(
                    hs_hbm.at[idx_v.at[pl.ds((c + 1) * ch, ch)]],
                    bufs[(c + 1) % 2], sems[(c + 1) % 2])
            pltpu.make_async_copy(
                hs_hbm.at[idx_v.at[pl.ds(c * ch, ch)]],
                bufs[c % 2], sems[c % 2]).wait()
            pltpu.sync_copy(bufs[c % 2], out_hbm.at[pl.ds(base + c * ch, ch)])

    return k(hs, padded_tok)


def _combine(y_pad, pos0, pos1):
    rows_per = _T // _NW
    ch = 8
    nch = rows_per // ch

    @functools.partial(
        pl.kernel, mesh=_sc_mesh(),
        out_type=jax.ShapeDtypeStruct((_T, _D), jnp.float32),
        scratch_types=[
            pltpu.VMEM((rows_per,), jnp.int32),
            pltpu.VMEM((rows_per,), jnp.int32),
            pltpu.VMEM((ch, _D), jnp.float32),
            pltpu.VMEM((ch, _D), jnp.float32),
            pltpu.VMEM((ch, _D), jnp.float32),
            pltpu.VMEM((ch, _D), jnp.float32),
            pltpu.SemaphoreType.DMA,
            pltpu.SemaphoreType.DMA,
            pltpu.SemaphoreType.DMA,
            pltpu.SemaphoreType.DMA,
        ],
    )
    def k(y_hbm, p0_hbm, p1_hbm, out_hbm, i0_v, i1_v, b0a, b1a, b0b, b1b,
          s0a, s1a, s0b, s1b):
        wid = lax.axis_index("s") * 2 + lax.axis_index("c")
        base = wid * rows_per
        pltpu.sync_copy(p0_hbm.at[pl.ds(base, rows_per)], i0_v)
        pltpu.sync_copy(p1_hbm.at[pl.ds(base, rows_per)], i1_v)
        b0 = (b0a, b0b)
        b1 = (b1a, b1b)
        s0 = (s0a, s0b)
        s1 = (s1a, s1b)

        def fire(c):
            pltpu.async_copy(y_hbm.at[i0_v.at[pl.ds(c * ch, ch)]],
                             b0[c % 2], s0[c % 2])
            pltpu.async_copy(y_hbm.at[i1_v.at[pl.ds(c * ch, ch)]],
                             b1[c % 2], s1[c % 2])

        fire(0)
        for c in range(nch):
            if c + 1 < nch:
                fire(c + 1)
            pltpu.make_async_copy(y_hbm.at[i0_v.at[pl.ds(c * ch, ch)]],
                                  b0[c % 2], s0[c % 2]).wait()
            pltpu.make_async_copy(y_hbm.at[i1_v.at[pl.ds(c * ch, ch)]],
                                  b1[c % 2], s1[c % 2]).wait()

            for r in range(ch):
                def addsl(j, c2, r=r):
                    sl = pl.ds(j * 16, 16)
                    b0[c % 2][r, sl] = b0[c % 2][r, sl] + b1[c % 2][r, sl]
                    return c2

                lax.fori_loop(0, _D // 16, addsl, 0)
            pltpu.sync_copy(b0[c % 2], out_hbm.at[pl.ds(base + c * ch, ch)])

    return k(y_pad, pos0, pos1)


def kernel(hidden_states, router_weight, e_score_correction_bias,
           gate_up_proj, down_proj):
    assert hidden_states.shape == (_T, _D)
    assert gate_up_proj.shape == (_E, _D, 2 * _I)
    hs = hidden_states.astype(jnp.float32)
    logits, i1, i2, w1, w2 = _run_router(hs, router_weight,
                                         e_score_correction_bias)
    padded_tok, w_pad, pos0, pos1, meta = _route_metadata(i1, i2, w1, w2)
    x_pad = _dispatch(hs, padded_tok)
    y_pad = _run_gmm(x_pad, gate_up_proj.astype(jnp.float32),
                     down_proj.astype(jnp.float32), w_pad, meta)
    final = _combine(y_pad, pos0, pos1)
    return final.reshape(-1), logits.reshape(-1)
